# baseline (speedup 1.0000x reference)
import functools

import jax
import jax.numpy as jnp
from jax import lax
from jax.experimental import pallas as pl
from jax.experimental.pallas import tpu as pltpu
from jax.experimental.pallas import tpu_sc as plsc

_C = 100000
_B = 1024
_IGNORE = 0
_SMOOTHING = 0.1
_CONF = 1.0 - _SMOOTHING
_EPS = _SMOOTHING / (_C - 1)

_NC = 2
_NS = 16
_L = 16
_NW = _NC * _NS
_BPW = _B // _NW

_ROWS_SC = 1024
_ROWS_TC = _B - _ROWS_SC
_DRW = _ROWS_SC // _NW

_TPR = 782
_TNB = 16


def _sc_body(logit_hbm, tgt, y_out, s_out, tgt_v, tiles_v, wtgt_v, dbuf,
             val_v, sacc_v, sem_g, sem_d):
    wid = lax.axis_index("s") * _NC + lax.axis_index("c")
    base = wid * _BPW
    lanes = lax.iota(jnp.int32, _L)

    pltpu.sync_copy(tgt.at[pl.ds(base, _BPW)], tgt_v)
    tvecs = [tgt_v[pl.ds(k * _L, _L)] for k in range(_BPW // _L)]
    tscal = []
    for i in range(_BPW):
        t = jnp.sum(jnp.where(lanes == (i % _L), tvecs[i // _L], 0))
        tscal.append(t)
        row8 = base + (i // 8) * 8
        col128 = (t // 128) * 128
        pltpu.make_async_copy(
            logit_hbm.at[pl.ds(row8, 8), pl.ds(col128, 128)],
            tiles_v.at[i],
            sem_g,
        ).start()

    dr0 = _ROWS_TC + wid * _DRW
    pltpu.sync_copy(tgt.at[pl.ds(dr0, _DRW)], wtgt_v)
    wfulls = [
        jnp.where(wtgt_v[pl.ds(k * _L, _L)] != _IGNORE, 1.0, 0.0)
        for k in range(_DRW // _L)
    ]
    wacc = jnp.zeros((_L,), jnp.float32)
    for tr in range(_DRW // 8):
        row8 = dr0 + 8 * tr
        wvs = []
        for r in range(8):
            idx = tr * 8 + r
            w_r = jnp.sum(jnp.where(lanes == (idx % _L), wfulls[idx // _L], 0.0))
            wvs.append(jnp.full((_L,), w_r, jnp.float32))

        def fire(j, slot, row8=row8):
            pltpu.make_async_copy(
                logit_hbm.at[pl.ds(row8, 8), pl.ds(j * 128, 128)],
                dbuf.at[slot],
                sem_d,
            ).start()

        def wait_one():
            pltpu.make_async_copy(
                logit_hbm.at[pl.ds(0, 8), pl.ds(0, 128)], dbuf.at[0], sem_d
            ).wait()

        def process(slot, a, nvec=8):
            out = list(a)
            for r in range(8):
                for v in range(nvec):
                    out[r] = out[r] + dbuf[slot, r, pl.ds(v * _L, _L)]
            return tuple(out)

        for b in range(_TNB):
            fire(b, b)
        accs = tuple(jnp.zeros((_L,), jnp.float32) for _ in range(8))

        def tbody(k, a):
            slot = lax.rem(k, _TNB)
            wait_one()
            a = process(slot, a)

            @pl.when(k + _TNB < _TPR)
            def _():
                fire(k + _TNB, slot)

            return a

        accs = lax.fori_loop(0, _TPR - 2, tbody, accs)
        wait_one()
        accs = process((_TPR - 2) % _TNB, accs)
        wait_one()
        accs = process((_TPR - 1) % _TNB, accs, nvec=2)
        for r in range(8):
            wacc = wacc + accs[r] * wvs[r]
    sacc_v[...] = wacc
    pltpu.sync_copy(sacc_v, s_out.at[pl.ds(wid * _L, _L)])

    for i in range(_BPW):
        pltpu.make_async_copy(
            logit_hbm.at[pl.ds(0, 8), pl.ds(0, 128)], tiles_v.at[i], sem_g
        ).wait()
    for k in range(_BPW // _L):
        yacc = jnp.zeros((_L,), jnp.float32)
        for j in range(_L):
            i = k * _L + j
            t = tscal[i]
            sub = (base + i) % 8
            l16 = ((t % 128) // 16) * 16
            vec = tiles_v[i, sub, pl.ds(l16, 16)]
            y = jnp.sum(jnp.where(lanes == (t % 16), vec, 0.0))
            y = jnp.where(t != _IGNORE, y, 0.0)
            yacc = jnp.where(lanes == j, y, yacc)
        val_v[pl.ds(k * _L, _L)] = yacc
    pltpu.sync_copy(val_v, y_out.at[pl.ds(base, _BPW)])


@functools.lru_cache(maxsize=1)
def _sc_kernel():
    return pl.kernel(
        _sc_body,
        mesh=plsc.VectorSubcoreMesh(core_axis_name="c", subcore_axis_name="s"),
        compiler_params=pltpu.CompilerParams(needs_layout_passes=False),
        out_type=(
            jax.ShapeDtypeStruct((_B,), jnp.float32),
            jax.ShapeDtypeStruct((_NW * _L,), jnp.float32),
        ),
        scratch_types=[
            pltpu.VMEM((_BPW,), jnp.int32),
            pltpu.VMEM((_BPW, 8, 128), jnp.float32),
            pltpu.VMEM((_DRW,), jnp.int32),
            pltpu.VMEM((_TNB, 8, 128), jnp.float32),
            pltpu.VMEM((_BPW,), jnp.float32),
            pltpu.VMEM((_L,), jnp.float32),
            pltpu.SemaphoreType.DMA,
            pltpu.SemaphoreType.DMA,
        ],
    )


def _combine_body(y_ref, ssc_ref, o_ref):
    o_ref[0, 0] = -(
        _EPS * jnp.sum(ssc_ref[...])
        + (_CONF - _EPS) * jnp.sum(y_ref[...])
    )


def kernel(logit, target):
    y, s_sc = _sc_kernel()(logit, target)
    out = pl.pallas_call(
        _combine_body,
        in_specs=[
            pl.BlockSpec((8, 128), lambda: (0, 0)),
            pl.BlockSpec((4, 128), lambda: (0, 0)),
        ],
        out_specs=pl.BlockSpec(memory_space=pltpu.SMEM),
        out_shape=jax.ShapeDtypeStruct((1, 1), jnp.float32),
    )(y.reshape(8, 128), s_sc.reshape(4, 128))
    return out[0, 0]


# speedup vs baseline: 1.0824x; 1.0824x over previous
import functools

import jax
import jax.numpy as jnp
from jax import lax
from jax.experimental import pallas as pl
from jax.experimental.pallas import tpu as pltpu
from jax.experimental.pallas import tpu_sc as plsc

_C = 100000
_B = 1024
_IGNORE = 0
_SMOOTHING = 0.1
_CONF = 1.0 - _SMOOTHING
_EPS = _SMOOTHING / (_C - 1)

_NC = 2
_NS = 16
_L = 16
_NW = _NC * _NS
_BPW = _B // _NW

_ROWS_SC = 512
_ROWS_TC = _B - _ROWS_SC
_DRW = _ROWS_SC // _NW

_TPR = 782
_TNB = 16


def _sc_body(logit_hbm, tgt, y_out, s_out, tgt_v, tiles_v, wtgt_v, dbuf,
             val_v, sacc_v, sem_g, sem_d):
    wid = lax.axis_index("s") * _NC + lax.axis_index("c")
    base = wid * _BPW
    lanes = lax.iota(jnp.int32, _L)

    pltpu.sync_copy(tgt.at[pl.ds(base, _BPW)], tgt_v)
    tvecs = [tgt_v[pl.ds(k * _L, _L)] for k in range(_BPW // _L)]
    tscal = []
    for i in range(_BPW):
        t = jnp.sum(jnp.where(lanes == (i % _L), tvecs[i // _L], 0))
        tscal.append(t)
        row8 = base + (i // 8) * 8
        col128 = (t // 128) * 128
        pltpu.make_async_copy(
            logit_hbm.at[pl.ds(row8, 8), pl.ds(col128, 128)],
            tiles_v.at[i],
            sem_g,
        ).start()

    dr0 = _ROWS_TC + wid * _DRW
    pltpu.sync_copy(tgt.at[pl.ds(dr0, _DRW)], wtgt_v)
    wfulls = [
        jnp.where(wtgt_v[pl.ds(k * _L, _L)] != _IGNORE, 1.0, 0.0)
        for k in range(_DRW // _L)
    ]
    wacc = jnp.zeros((_L,), jnp.float32)
    for tr in range(_DRW // 8):
        row8 = dr0 + 8 * tr
        wvs = []
        for r in range(8):
            idx = tr * 8 + r
            w_r = jnp.sum(jnp.where(lanes == (idx % _L), wfulls[idx // _L], 0.0))
            wvs.append(jnp.full((_L,), w_r, jnp.float32))

        def fire(j, slot, row8=row8):
            pltpu.make_async_copy(
                logit_hbm.at[pl.ds(row8, 8), pl.ds(j * 128, 128)],
                dbuf.at[slot],
                sem_d,
            ).start()

        def wait_one():
            pltpu.make_async_copy(
                logit_hbm.at[pl.ds(0, 8), pl.ds(0, 128)], dbuf.at[0], sem_d
            ).wait()

        def process(slot, a, nvec=8):
            out = list(a)
            for r in range(8):
                for v in range(nvec):
                    out[r] = out[r] + dbuf[slot, r, pl.ds(v * _L, _L)]
            return tuple(out)

        for b in range(_TNB):
            fire(b, b)
        accs = tuple(jnp.zeros((_L,), jnp.float32) for _ in range(8))

        def tbody(k, a):
            slot = lax.rem(k, _TNB)
            wait_one()
            a = process(slot, a)

            @pl.when(k + _TNB < _TPR)
            def _():
                fire(k + _TNB, slot)

            return a

        accs = lax.fori_loop(0, _TPR - 2, tbody, accs)
        wait_one()
        accs = process((_TPR - 2) % _TNB, accs)
        wait_one()
        accs = process((_TPR - 1) % _TNB, accs, nvec=2)
        for r in range(8):
            wacc = wacc + accs[r] * wvs[r]
    sacc_v[...] = wacc
    pltpu.sync_copy(sacc_v, s_out.at[pl.ds(wid * _L, _L)])

    for i in range(_BPW):
        pltpu.make_async_copy(
            logit_hbm.at[pl.ds(0, 8), pl.ds(0, 128)], tiles_v.at[i], sem_g
        ).wait()
    for k in range(_BPW // _L):
        yacc = jnp.zeros((_L,), jnp.float32)
        for j in range(_L):
            i = k * _L + j
            t = tscal[i]
            sub = (base + i) % 8
            l16 = ((t % 128) // 16) * 16
            vec = tiles_v[i, sub, pl.ds(l16, 16)]
            y = jnp.sum(jnp.where(lanes == (t % 16), vec, 0.0))
            y = jnp.where(t != _IGNORE, y, 0.0)
            yacc = jnp.where(lanes == j, y, yacc)
        val_v[pl.ds(k * _L, _L)] = yacc
    pltpu.sync_copy(val_v, y_out.at[pl.ds(base, _BPW)])


@functools.lru_cache(maxsize=1)
def _sc_kernel():
    return pl.kernel(
        _sc_body,
        mesh=plsc.VectorSubcoreMesh(core_axis_name="c", subcore_axis_name="s"),
        compiler_params=pltpu.CompilerParams(needs_layout_passes=False),
        out_type=(
            jax.ShapeDtypeStruct((_B,), jnp.float32),
            jax.ShapeDtypeStruct((_NW * _L,), jnp.float32),
        ),
        scratch_types=[
            pltpu.VMEM((_BPW,), jnp.int32),
            pltpu.VMEM((_BPW, 8, 128), jnp.float32),
            pltpu.VMEM((_DRW,), jnp.int32),
            pltpu.VMEM((_TNB, 8, 128), jnp.float32),
            pltpu.VMEM((_BPW,), jnp.float32),
            pltpu.VMEM((_L,), jnp.float32),
            pltpu.SemaphoreType.DMA,
            pltpu.SemaphoreType.DMA,
        ],
    )


_BR = 16
_TCG = _ROWS_TC // (2 * _BR)


def _tc_reduce_body(tgt1_ref, tgt2_ref, x1_ref, x2_ref, o_ref):
    j = pl.program_id(0)

    @pl.when(j == 0)
    def _():
        o_ref[0, 0] = 0.0

    w1 = (tgt1_ref[...] != _IGNORE).astype(jnp.float32)
    w2 = (tgt2_ref[...] != _IGNORE).astype(jnp.float32)
    o_ref[0, 0] += _EPS * (jnp.sum(x1_ref[...] * w1) + jnp.sum(x2_ref[...] * w2))


def _combine_body(stc_ref, y_ref, ssc_ref, o_ref):
    o_ref[0, 0] = -(
        stc_ref[0, 0]
        + _EPS * jnp.sum(ssc_ref[...])
        + (_CONF - _EPS) * jnp.sum(y_ref[...])
    )


def kernel(logit, target):
    y, s_sc = _sc_kernel()(logit, target)
    tgt2d = target.reshape(_B, 1)
    s_tc = pl.pallas_call(
        _tc_reduce_body,
        grid=(_TCG,),
        in_specs=[
            pl.BlockSpec((_BR, 1), lambda j: (j, 0)),
            pl.BlockSpec((_BR, 1), lambda j: (j + _TCG, 0)),
            pl.BlockSpec((_BR, _C), lambda j: (j, 0)),
            pl.BlockSpec((_BR, _C), lambda j: (j + _TCG, 0)),
        ],
        out_specs=pl.BlockSpec(memory_space=pltpu.SMEM),
        out_shape=jax.ShapeDtypeStruct((1, 1), jnp.float32),
    )(tgt2d, tgt2d, logit, logit)
    out = pl.pallas_call(
        _combine_body,
        in_specs=[
            pl.BlockSpec(memory_space=pltpu.SMEM),
            pl.BlockSpec((8, 128), lambda: (0, 0)),
            pl.BlockSpec((4, 128), lambda: (0, 0)),
        ],
        out_specs=pl.BlockSpec(memory_space=pltpu.SMEM),
        out_shape=jax.ShapeDtypeStruct((1, 1), jnp.float32),
    )(s_tc, y.reshape(8, 128), s_sc.reshape(4, 128))
    return out[0, 0]


# speedup vs baseline: 1.3507x; 1.2479x over previous
import functools

import jax
import jax.numpy as jnp
from jax import lax
from jax.experimental import pallas as pl
from jax.experimental.pallas import tpu as pltpu
from jax.experimental.pallas import tpu_sc as plsc

_C = 100000
_B = 1024
_IGNORE = 0
_SMOOTHING = 0.1
_CONF = 1.0 - _SMOOTHING
_EPS = _SMOOTHING / (_C - 1)

_NC = 2
_NS = 16
_L = 16
_NW = _NC * _NS
_BPW = _B // _NW

_CTC = 36000
_CSC = _C - _CTC
_NTCOL = _B // 128
_NQ = _NW // _NTCOL
_CPW = _CSC // _NQ
_H = 64
_NCH = _CPW // _H
_DNB = 2

_BC = 2000
_TCG = _CTC // (2 * _BC)


def _sc_body(xt_hbm, tgt, y_out, s_out, tgt_v, tiles_v, wtgt_v, dbuf,
             val_v, sacc_v, sem_g, sem_d):
    wid = lax.axis_index("s") * _NC + lax.axis_index("c")
    base = wid * _BPW
    lanes = lax.iota(jnp.int32, _L)

    pltpu.sync_copy(tgt.at[pl.ds(base, _BPW)], tgt_v)
    tvecs = [tgt_v[pl.ds(k * _L, _L)] for k in range(_BPW // _L)]
    tscal = []
    for i in range(_BPW):
        t = jnp.sum(jnp.where(lanes == (i % _L), tvecs[i // _L], 0))
        tscal.append(t)
        trow8 = (t // 8) * 8
        bcol128 = ((base + i) // 128) * 128
        pltpu.make_async_copy(
            xt_hbm.at[pl.ds(trow8, 8), pl.ds(bcol128, 128)],
            tiles_v.at[i],
            sem_g,
        ).start()

    tcol = lax.rem(wid, _NTCOL)
    q = wid // _NTCOL
    c0 = _CTC + q * _CPW
    bcol = tcol * 128

    def fire(ch, slot):
        pltpu.make_async_copy(
            xt_hbm.at[pl.ds(c0 + ch * _H, _H), pl.ds(bcol, 128)],
            dbuf.at[slot],
            sem_d,
        ).start()

    def wait_one():
        pltpu.make_async_copy(
            xt_hbm.at[pl.ds(0, _H), pl.ds(0, 128)], dbuf.at[0], sem_d
        ).wait()

    for b in range(_DNB):
        fire(b, b)
    accs = tuple(jnp.zeros((_L,), jnp.float32) for _ in range(8))

    def chunk_body(ch, a):
        slot = lax.rem(ch, _DNB)
        wait_one()
        out = list(a)
        for r in range(_H):
            for g in range(8):
                out[g] = out[g] + dbuf[slot, r, pl.ds(g * _L, _L)]
        a = tuple(out)

        @pl.when(ch + _DNB < _NCH)
        def _():
            fire(ch + _DNB, slot)

        return a

    accs = lax.fori_loop(0, _NCH, chunk_body, accs)

    pltpu.sync_copy(tgt.at[pl.ds(bcol, 128)], wtgt_v)
    for g in range(8):
        w = jnp.where(wtgt_v[pl.ds(g * _L, _L)] != _IGNORE, 1.0, 0.0)
        sacc_v[pl.ds(g * _L, _L)] = accs[g] * w
    pltpu.sync_copy(sacc_v, s_out.at[pl.ds(wid * 128, 128)])

    for i in range(_BPW):
        pltpu.make_async_copy(
            xt_hbm.at[pl.ds(0, 8), pl.ds(0, 128)], tiles_v.at[i], sem_g
        ).wait()
    for k in range(_BPW // _L):
        yacc = jnp.zeros((_L,), jnp.float32)
        for j in range(_L):
            i = k * _L + j
            t = tscal[i]
            sub = lax.rem(t, 8)
            l16 = (lax.rem(base + i, 128) // _L) * _L
            vec = tiles_v[i, sub, pl.ds(l16, _L)]
            y = jnp.sum(jnp.where(lanes == (i % _L), vec, 0.0))
            y = jnp.where(t != _IGNORE, y, 0.0)
            yacc = jnp.where(lanes == j, y, yacc)
        val_v[pl.ds(k * _L, _L)] = yacc
    pltpu.sync_copy(val_v, y_out.at[pl.ds(base, _BPW)])


@functools.lru_cache(maxsize=1)
def _sc_kernel():
    return pl.kernel(
        _sc_body,
        mesh=plsc.VectorSubcoreMesh(core_axis_name="c", subcore_axis_name="s"),
        compiler_params=pltpu.CompilerParams(needs_layout_passes=False),
        out_type=(
            jax.ShapeDtypeStruct((_B,), jnp.float32),
            jax.ShapeDtypeStruct((_NW * 128,), jnp.float32),
        ),
        scratch_types=[
            pltpu.VMEM((_BPW,), jnp.int32),
            pltpu.VMEM((_BPW, 8, 128), jnp.float32),
            pltpu.VMEM((128,), jnp.int32),
            pltpu.VMEM((_DNB, _H, 128), jnp.float32),
            pltpu.VMEM((_BPW,), jnp.float32),
            pltpu.VMEM((128,), jnp.float32),
            pltpu.SemaphoreType.DMA,
            pltpu.SemaphoreType.DMA,
        ],
    )


def _tc_reduce_body(tgt_ref, x1_ref, x2_ref, o_ref):
    j = pl.program_id(0)

    @pl.when(j == 0)
    def _():
        o_ref[0, 0] = 0.0

    w = (tgt_ref[...] != _IGNORE).astype(jnp.float32)
    o_ref[0, 0] += _EPS * (jnp.sum(x1_ref[...] * w) + jnp.sum(x2_ref[...] * w))


def _combine_body(stc_ref, y_ref, ssc_ref, o_ref):
    o_ref[0, 0] = -(
        stc_ref[0, 0]
        + _EPS * jnp.sum(ssc_ref[...])
        + (_CONF - _EPS) * jnp.sum(y_ref[...])
    )


def kernel(logit, target):
    xt = logit.T
    y, s_sc = _sc_kernel()(xt, target)
    s_tc = pl.pallas_call(
        _tc_reduce_body,
        grid=(_TCG,),
        in_specs=[
            pl.BlockSpec((1, _B), lambda j: (0, 0)),
            pl.BlockSpec((_BC, _B), lambda j: (j, 0)),
            pl.BlockSpec((_BC, _B), lambda j: (j + _TCG, 0)),
        ],
        out_specs=pl.BlockSpec(memory_space=pltpu.SMEM),
        out_shape=jax.ShapeDtypeStruct((1, 1), jnp.float32),
    )(target.reshape(1, _B), xt, xt)
    out = pl.pallas_call(
        _combine_body,
        in_specs=[
            pl.BlockSpec(memory_space=pltpu.SMEM),
            pl.BlockSpec((8, 128), lambda: (0, 0)),
            pl.BlockSpec((32, 128), lambda: (0, 0)),
        ],
        out_specs=pl.BlockSpec(memory_space=pltpu.SMEM),
        out_shape=jax.ShapeDtypeStruct((1, 1), jnp.float32),
    )(s_tc, y.reshape(8, 128), s_sc.reshape(32, 128))
    return out[0, 0]


# speedup vs baseline: 3.6439x; 2.6977x over previous
import functools

import jax
import jax.numpy as jnp
from jax import lax
from jax.experimental import pallas as pl
from jax.experimental.pallas import tpu as pltpu
from jax.experimental.pallas import tpu_sc as plsc

_C = 100000
_B = 1024
_IGNORE = 0
_SMOOTHING = 0.1
_CONF = 1.0 - _SMOOTHING
_EPS = _SMOOTHING / (_C - 1)

_NC = 2
_NS = 16
_L = 16
_NW = _NC * _NS
_BPW = _B // _NW

_CTC = 74400
_CSC = _C - _CTC
_CPW = _CSC // _NW
_H = 32
_NCH = _CPW // _H
_DNB = 2
_NG = _B // _L

_BC = 2048
_TCG = (_CTC + 2 * _BC - 1) // (2 * _BC)


def _sc_body(xt_hbm, tgt, y_out, s_out, tgt_v, tiles_v, wtgt_v, dbuf,
             val_v, sacc_v, sem_g, sem_d):
    wid = lax.axis_index("s") * _NC + lax.axis_index("c")
    base = wid * _BPW
    lanes = lax.iota(jnp.int32, _L)

    pltpu.sync_copy(tgt.at[pl.ds(base, _BPW)], tgt_v)
    tvecs = [tgt_v[pl.ds(k * _L, _L)] for k in range(_BPW // _L)]
    tscal = []
    for i in range(_BPW):
        t = jnp.sum(jnp.where(lanes == (i % _L), tvecs[i // _L], 0))
        tscal.append(t)
        trow8 = (t // 8) * 8
        bcol128 = ((base + i) // 128) * 128
        pltpu.make_async_copy(
            xt_hbm.at[pl.ds(trow8, 8), pl.ds(bcol128, 128)],
            tiles_v.at[i],
            sem_g,
        ).start()

    c0 = _CTC + wid * _CPW

    def zero_g(g, _):
        sacc_v[pl.ds(g * _L, _L)] = jnp.zeros((_L,), jnp.float32)
        return 0

    lax.fori_loop(0, _NG, zero_g, 0)

    def fire(ch, slot):
        pltpu.make_async_copy(
            xt_hbm.at[pl.ds(c0 + ch * _H, _H), :],
            dbuf.at[slot],
            sem_d,
        ).start()

    def wait_one():
        pltpu.make_async_copy(
            xt_hbm.at[pl.ds(0, _H), :], dbuf.at[0], sem_d
        ).wait()

    for b in range(_DNB):
        fire(b, b)

    def chunk_body(ch, _):
        slot = lax.rem(ch, _DNB)
        wait_one()

        def gbody(g, __):
            gs = pl.ds(g * _L, _L)
            vals = [dbuf[slot, r, gs] for r in range(_H)]
            while len(vals) > 1:
                vals = [vals[i] + vals[i + 1] for i in range(0, len(vals), 2)]
            sacc_v[gs] = sacc_v[gs] + vals[0]
            return 0

        lax.fori_loop(0, _NG, gbody, 0)

        @pl.when(ch + _DNB < _NCH)
        def _():
            fire(ch + _DNB, slot)

        return 0

    lax.fori_loop(0, _NCH, chunk_body, 0)

    pltpu.sync_copy(tgt, wtgt_v)

    def mask_g(g, _):
        gs = pl.ds(g * _L, _L)
        w = jnp.where(wtgt_v[gs] != _IGNORE, 1.0, 0.0)
        sacc_v[gs] = sacc_v[gs] * w
        return 0

    lax.fori_loop(0, _NG, mask_g, 0)
    pltpu.sync_copy(sacc_v, s_out.at[pl.ds(wid * _B, _B)])

    for i in range(_BPW):
        pltpu.make_async_copy(
            xt_hbm.at[pl.ds(0, 8), pl.ds(0, 128)], tiles_v.at[i], sem_g
        ).wait()
    for k in range(_BPW // _L):
        yacc = jnp.zeros((_L,), jnp.float32)
        for j in range(_L):
            i = k * _L + j
            t = tscal[i]
            sub = lax.rem(t, 8)
            l16 = (lax.rem(base + i, 128) // _L) * _L
            vec = tiles_v[i, sub, pl.ds(l16, _L)]
            y = jnp.sum(jnp.where(lanes == (i % _L), vec, 0.0))
            y = jnp.where(t != _IGNORE, y, 0.0)
            yacc = jnp.where(lanes == j, y, yacc)
        val_v[pl.ds(k * _L, _L)] = yacc
    pltpu.sync_copy(val_v, y_out.at[pl.ds(base, _BPW)])


@functools.lru_cache(maxsize=1)
def _sc_kernel():
    return pl.kernel(
        _sc_body,
        mesh=plsc.VectorSubcoreMesh(core_axis_name="c", subcore_axis_name="s"),
        compiler_params=pltpu.CompilerParams(needs_layout_passes=False),
        out_type=(
            jax.ShapeDtypeStruct((_B,), jnp.float32),
            jax.ShapeDtypeStruct((_NW * _B,), jnp.float32),
        ),
        scratch_types=[
            pltpu.VMEM((_BPW,), jnp.int32),
            pltpu.VMEM((_BPW, 8, 128), jnp.float32),
            pltpu.VMEM((_B,), jnp.int32),
            pltpu.VMEM((_DNB, _H, _B), jnp.float32),
            pltpu.VMEM((_BPW,), jnp.float32),
            pltpu.VMEM((_B,), jnp.float32),
            pltpu.SemaphoreType.DMA,
            pltpu.SemaphoreType.DMA,
        ],
    )


def _tc_reduce_body(tgt_ref, x1_ref, x2_ref, o_ref):
    j = pl.program_id(0)

    @pl.when(j == 0)
    def _():
        o_ref[0, 0] = 0.0

    w = (tgt_ref[...] != _IGNORE).astype(jnp.float32)
    rows2 = (j + _TCG) * _BC + lax.broadcasted_iota(jnp.int32, (_BC, _B), 0)
    x2 = jnp.where(rows2 < _CTC, x2_ref[...], 0.0)
    o_ref[0, 0] += _EPS * (jnp.sum(x1_ref[...] * w) + jnp.sum(x2 * w))


def _combine_body(stc_ref, y_ref, ssc_ref, o_ref):
    o_ref[0, 0] = -(
        stc_ref[0, 0]
        + _EPS * jnp.sum(ssc_ref[...])
        + (_CONF - _EPS) * jnp.sum(y_ref[...])
    )


def kernel(logit, target):
    xt = logit.T
    y, s_sc = _sc_kernel()(xt, target)
    s_tc = pl.pallas_call(
        _tc_reduce_body,
        grid=(_TCG,),
        in_specs=[
            pl.BlockSpec((1, _B), lambda j: (0, 0)),
            pl.BlockSpec((_BC, _B), lambda j: (j, 0)),
            pl.BlockSpec((_BC, _B), lambda j: (j + _TCG, 0)),
        ],
        out_specs=pl.BlockSpec(memory_space=pltpu.SMEM),
        out_shape=jax.ShapeDtypeStruct((1, 1), jnp.float32),
    )(target.reshape(1, _B), xt, xt)
    out = pl.pallas_call(
        _combine_body,
        in_specs=[
            pl.BlockSpec(memory_space=pltpu.SMEM),
            pl.BlockSpec((8, 128), lambda: (0, 0)),
            pl.BlockSpec((_NW * 8, 128), lambda: (0, 0)),
        ],
        out_specs=pl.BlockSpec(memory_space=pltpu.SMEM),
        out_shape=jax.ShapeDtypeStruct((1, 1), jnp.float32),
    )(s_tc, y.reshape(8, 128), s_sc.reshape(_NW * 8, 128))
    return out[0, 0]
